# baseline (speedup 1.0000x reference)
import functools

import jax
import jax.numpy as jnp
from jax import lax
from jax.experimental import pallas as pl
from jax.experimental.pallas import tpu as pltpu
from jax.experimental.pallas import tpu_sc as plsc

N_TOK = 32768
N_EXPERTS = 64
K = 8
BLOCK_ROWS = 1024

NC, NS, NLANES = 2, 16, 16
NW = NC * NS
TPW = N_TOK // NW
CHUNK = 256


def _logits_kernel(x_ref, w_ref, b_ref, out_ref):
    out_ref[...] = (
        jnp.dot(x_ref[...], w_ref[...], preferred_element_type=jnp.float32)
        + b_ref[...]
    )


def _sc_router(logits_hbm, out_hbm, idx_hbm, in_v, out_v, idx_v):
    wid = lax.axis_index("s") * NC + lax.axis_index("c")
    base = wid * TPW
    lane = lax.iota(jnp.int32, 16)
    intmin = jnp.int32(-(2**31))
    perms = [lane ^ jnp.int32(s) for s in (8, 4, 2, 1)]

    def _shuffle(v, idx):
        return v.at[idx].get(mode="promise_in_bounds")

    def _allmax(v):
        for idx in perms:
            v = jnp.maximum(v, _shuffle(v, idx))
        return v

    def _allsum(v):
        for idx in perms:
            v = v + _shuffle(v, idx)
        return v

    def pair_body(p, carry):
        idxacc = jnp.zeros((16,), jnp.int32)
        for h in range(2):
            t = 2 * p + h
            keys = []
            for e in range(4):
                v = in_v[pl.ds(t * N_EXPERTS + e * 16, 16)]
                raw = lax.bitcast_convert_type(v, jnp.int32)
                kk = jnp.where(raw < 0, raw ^ jnp.int32(0x7FFFFFFF), raw)
                col = jnp.int32(N_EXPERTS - 1) - (jnp.int32(e * 16) + lane)
                keys.append((kk & jnp.int32(~63)) | col)
            work = list(keys)
            m0 = None
            for r in range(K):
                m = _allmax(
                    jnp.maximum(
                        jnp.maximum(work[0], work[1]), jnp.maximum(work[2], work[3])
                    )
                )
                if m0 is None:
                    m0 = m
                idxr = jnp.int32(N_EXPERTS - 1) - (m & jnp.int32(63))
                idxacc = idxacc + jnp.where(
                    lane == jnp.int32(8 * h + r), idxr, jnp.int32(0)
                )
                for e in range(4):
                    work[e] = jnp.where(work[e] == m, intmin, work[e])
            vm = jnp.where(m0 < 0, m0 ^ jnp.int32(0x7FFFFFFF), m0)
            vmax = lax.bitcast_convert_type(vm, jnp.float32)
            es = []
            for e in range(4):
                uk = jnp.where(keys[e] < 0, keys[e] ^ jnp.int32(0x7FFFFFFF), keys[e])
                val = lax.bitcast_convert_type(uk, jnp.float32)
                es.append(
                    jnp.where(work[e] == intmin, jnp.exp(val - vmax), jnp.float32(0.0))
                )
            denom = _allsum(es[0] + es[1] + es[2] + es[3])
            for e in range(4):
                out_v[pl.ds(t * N_EXPERTS + e * 16, 16)] = es[e] / denom
        idx_v[pl.ds(p * 16, 16)] = idxacc
        return carry

    for c in range(TPW // CHUNK):
        tok0 = base + c * CHUNK
        pltpu.sync_copy(
            logits_hbm.at[pl.ds(tok0 * N_EXPERTS, CHUNK * N_EXPERTS)], in_v
        )
        lax.fori_loop(0, CHUNK // 2, pair_body, 0)
        pltpu.sync_copy(
            out_v, out_hbm.at[pl.ds(tok0 * N_EXPERTS, CHUNK * N_EXPERTS)]
        )
        pltpu.sync_copy(idx_v, idx_hbm.at[pl.ds(tok0 * K, CHUNK * K)])


@jax.jit
def kernel(x, W, b):
    n, d = x.shape
    logits = pl.pallas_call(
        _logits_kernel,
        grid=(n // BLOCK_ROWS,),
        in_specs=[
            pl.BlockSpec((BLOCK_ROWS, d), lambda i: (i, 0)),
            pl.BlockSpec((d, N_EXPERTS), lambda i: (0, 0)),
            pl.BlockSpec((1, N_EXPERTS), lambda i: (0, 0)),
        ],
        out_specs=pl.BlockSpec((BLOCK_ROWS, N_EXPERTS), lambda i: (i, 0)),
        out_shape=jax.ShapeDtypeStruct((n, N_EXPERTS), jnp.float32),
        compiler_params=pltpu.CompilerParams(
            dimension_semantics=("parallel",),
        ),
    )(x, W, b.reshape(1, N_EXPERTS))

    router = functools.partial(
        pl.kernel,
        mesh=plsc.VectorSubcoreMesh(core_axis_name="c", subcore_axis_name="s"),
        out_type=[
            jax.ShapeDtypeStruct((n * N_EXPERTS,), jnp.float32),
            jax.ShapeDtypeStruct((n * K,), jnp.int32),
        ],
        scratch_types=[
            pltpu.VMEM((CHUNK * N_EXPERTS,), jnp.float32),
            pltpu.VMEM((CHUNK * N_EXPERTS,), jnp.float32),
            pltpu.VMEM((CHUNK * K,), jnp.int32),
        ],
    )(_sc_router)
    out_flat, idx_flat = router(logits.reshape(-1))
    return (out_flat.reshape(n, N_EXPERTS), idx_flat.reshape(n, K))


# speedup vs baseline: 1.6997x; 1.6997x over previous
import functools

import jax
import jax.numpy as jnp
from jax.experimental import pallas as pl
from jax.experimental.pallas import tpu as pltpu

N_EXPERTS = 64
K = 8
BLOCK_ROWS = 1024


def _router_kernel(x_ref, w_ref, b_ref, out_ref, idx_ref):
    logits = (
        jnp.dot(x_ref[...], w_ref[...], preferred_element_type=jnp.float32)
        + b_ref[...]
    )
    col = jax.lax.broadcasted_iota(jnp.int32, logits.shape, 1)
    raw = jax.lax.bitcast_convert_type(logits, jnp.int32)
    key = jnp.where(raw < 0, raw ^ jnp.int32(0x7FFFFFFF), raw)
    keys = (key & jnp.int32(~63)) | (jnp.int32(N_EXPERTS - 1) - col)
    kt = keys.T
    work = kt
    idx_rows = []
    m0 = None
    for _ in range(K):
        m = jnp.max(work, axis=0, keepdims=True)
        if m0 is None:
            m0 = m
        idx_rows.append(jnp.int32(N_EXPERTS - 1) - (m & jnp.int32(63)))
        work = jnp.where(work == m, jnp.int32(-(2**31)), work)
    sel = work == jnp.int32(-(2**31))
    vt = jax.lax.bitcast_convert_type(
        jnp.where(kt < 0, kt ^ jnp.int32(0x7FFFFFFF), kt), jnp.float32
    )
    vmax = jax.lax.bitcast_convert_type(
        jnp.where(m0 < 0, m0 ^ jnp.int32(0x7FFFFFFF), m0), jnp.float32
    )
    e = jnp.where(sel, jnp.exp(vt - vmax), 0.0)
    out_ref[...] = (e / jnp.sum(e, axis=0, keepdims=True)).T
    idx_ref[...] = jnp.concatenate(idx_rows, axis=0).T


@jax.jit
def kernel(x, W, b):
    n, d = x.shape
    grid = (n // BLOCK_ROWS,)
    out, idx = pl.pallas_call(
        _router_kernel,
        grid=grid,
        in_specs=[
            pl.BlockSpec((BLOCK_ROWS, d), lambda i: (i, 0)),
            pl.BlockSpec((d, N_EXPERTS), lambda i: (0, 0)),
            pl.BlockSpec((1, N_EXPERTS), lambda i: (0, 0)),
        ],
        out_specs=[
            pl.BlockSpec((BLOCK_ROWS, N_EXPERTS), lambda i: (i, 0)),
            pl.BlockSpec((BLOCK_ROWS, K), lambda i: (i, 0)),
        ],
        out_shape=[
            jax.ShapeDtypeStruct((n, N_EXPERTS), jnp.float32),
            jax.ShapeDtypeStruct((n, K), jnp.int32),
        ],
        compiler_params=pltpu.CompilerParams(
            dimension_semantics=("parallel",),
        ),
    )(x, W, b.reshape(1, N_EXPERTS))
    return (out, idx)
